# baseline (speedup 1.0000x reference)
import functools

import jax
import jax.numpy as jnp
from jax import lax
from jax.experimental import pallas as pl
from jax.experimental.pallas import tpu as pltpu
from jax.experimental.pallas import tpu_sc as plsc

NC = 2
NS = 16
LANES = 16
NW = NC * NS


_EB = 8192


def _coeff_body(n_edges, d_ref, c_ref, w_ref, o_ref):
    i = pl.program_id(0)
    e = (i * _EB + jax.lax.broadcasted_iota(jnp.int32, (_EB // 128, 128), 0)
         * 128 + jax.lax.broadcasted_iota(jnp.int32, (_EB // 128, 128), 1))
    v = (d_ref[...] * w_ref[0] + c_ref[...] * w_ref[1] + w_ref[2]) * w_ref[3]
    v = jnp.where(e < n_edges, v.reshape(_EB // 128, 128), 0.0)
    o_ref[...] = v.reshape(_EB // CHUNK, CHUNK)


def _edge_coeff(direction, cfg, wvec, n_edges, e_pad):
    body = functools.partial(_coeff_body, n_edges)
    return pl.pallas_call(
        body,
        grid=(e_pad // _EB,),
        out_shape=jax.ShapeDtypeStruct((e_pad // CHUNK, CHUNK), jnp.float32),
        in_specs=[
            pl.BlockSpec((1, _EB), lambda i: (0, i)),
            pl.BlockSpec((1, _EB), lambda i: (0, i)),
            pl.BlockSpec(memory_space=pltpu.SMEM),
        ],
        out_specs=pl.BlockSpec((_EB // CHUNK, CHUNK), lambda i: (i, 0)),
    )(direction, cfg, wvec)


def _idx_body(n_edges, n_nodes, n_acc_pad, x_ref, s_ref, t_ref):
    i = pl.program_id(0)
    x = x_ref[...]
    e = (i * _EB + jax.lax.broadcasted_iota(jnp.int32, (_EB // 64, 64), 0)
         * 64 + jax.lax.broadcasted_iota(jnp.int32, (_EB // 64, 64), 1))
    jj = jax.lax.broadcasted_iota(jnp.int32, (128, 64), 0)
    kk = jax.lax.broadcasted_iota(jnp.int32, (128, 64), 1)
    xf = x.astype(jnp.float32)
    src = jnp.dot(xf, (jj == 2 * kk).astype(jnp.float32),
                  preferred_element_type=jnp.float32).astype(jnp.int32)
    tgt = jnp.dot(xf, (jj == 2 * kk + 1).astype(jnp.float32),
                  preferred_element_type=jnp.float32).astype(jnp.int32)
    src = jnp.where(e < n_edges, src, e % n_nodes)
    tgt = jnp.where(e < n_edges, tgt, n_nodes + e % (n_acc_pad - n_nodes))
    s_ref[...] = src.reshape(_EB // CHUNK, CHUNK)
    t_ref[...] = tgt.reshape(_EB // CHUNK, CHUNK)


def _edge_idx(cfg2, n_edges, n_nodes, n_acc_pad, e_pad):
    body = functools.partial(_idx_body, n_edges, n_nodes, n_acc_pad)
    shp = jax.ShapeDtypeStruct((e_pad // CHUNK, CHUNK), jnp.int32)
    return pl.pallas_call(
        body,
        grid=(e_pad // _EB,),
        out_shape=(shp, shp),
        in_specs=[pl.BlockSpec((_EB // 64, 128), lambda i: (i, 0))],
        out_specs=(pl.BlockSpec((_EB // CHUNK, CHUNK), lambda i: (i, 0)),
                   pl.BlockSpec((_EB // CHUNK, CHUNK), lambda i: (i, 0))),
    )(cfg2)


CHUNK = 64
NBUF = 4
SBUF = 2
IB = 8


def _make_sc_kernel(n_pad, d_feat, e_pad):
    cpw = e_pad // (NW * CHUNK)
    nu = cpw // IB
    rows_per_tile = n_pad // NS
    mesh = plsc.VectorSubcoreMesh(
        core_axis_name="c", subcore_axis_name="s", num_cores=NC, num_subcores=NS
    )

    @functools.partial(
        pl.kernel,
        out_type=jax.ShapeDtypeStruct((NC, n_pad, d_feat), jnp.float32),
        mesh=mesh,
        compiler_params=pltpu.CompilerParams(needs_layout_passes=False, use_tc_tiling_on_sc=False),
        scratch_types=[
            pltpu.VMEM_SHARED((n_pad, d_feat), jnp.float32),
            [pltpu.VMEM((CHUNK,), jnp.int32) for _ in range(IB)],
            [pltpu.VMEM((CHUNK,), jnp.int32) for _ in range(IB)],
            [pltpu.VMEM((CHUNK,), jnp.float32) for _ in range(IB)],
            [pltpu.VMEM((CHUNK, d_feat // 2), jnp.int32) for _ in range(NBUF)],
            [pltpu.VMEM((CHUNK, d_feat), jnp.float32) for _ in range(SBUF)],
            [pltpu.SemaphoreType.DMA for _ in range(NBUF)],
            [pltpu.SemaphoreType.DMA for _ in range(SBUF)],
            [pltpu.SemaphoreType.DMA for _ in range(IB)],
            pltpu.SemaphoreType.DMA,
        ],
    )
    def sc_kernel(trace_hbm, src_hbm, tgt_hbm, coeff_hbm, zeros_hbm,
                  out_hbm, acc, sidx, tidx, cbuf, rows, srows,
                  gsem, ssem, isem, zsem):
        cid = lax.axis_index("c")
        sid = lax.axis_index("s")
        wid = cid * NS + sid
        c0 = wid * cpw

        def _idx_start(c, sl):
            pltpu.async_copy(src_hbm.at[c0 + c], sidx[sl], isem[sl])
            pltpu.async_copy(tgt_hbm.at[c0 + c], tidx[sl], isem[sl])
            pltpu.async_copy(coeff_hbm.at[c0 + c], cbuf[sl], isem[sl])

        def _idx_wait(c, sl):
            pltpu.make_async_copy(src_hbm.at[c0 + c], sidx[sl], isem[sl]).wait()
            pltpu.make_async_copy(tgt_hbm.at[c0 + c], tidx[sl], isem[sl]).wait()
            pltpu.make_async_copy(coeff_hbm.at[c0 + c], cbuf[sl], isem[sl]).wait()

        def _gather_start(sl, b):
            pltpu.async_copy(trace_hbm.at[sidx[sl]], rows[b], gsem[b])

        def _gather_wait(sl, b):
            pltpu.make_async_copy(
                trace_hbm.at[sidx[sl]], rows[b], gsem[b]).wait()

        def _scatter_start(sl, s):
            pltpu.async_copy(srows[s], acc.at[tidx[sl]], ssem[s], add=True)

        def _scatter_wait(sl, s):
            pltpu.make_async_copy(srows[s], acc.at[tidx[sl]], ssem[s]).wait()

        row0 = sid * rows_per_tile
        for c in range(6):
            _idx_start(c, c)
        for c in range(3):
            _idx_wait(c, c)
            _gather_start(c, c)

        pltpu.async_copy(zeros_hbm.at[pl.ds(row0, rows_per_tile)],
                         acc.at[pl.ds(row0, rows_per_tile)], zsem)
        pltpu.make_async_copy(zeros_hbm.at[pl.ds(row0, rows_per_tile)],
                              acc.at[pl.ds(row0, rows_per_tile)], zsem).wait()
        plsc.subcore_barrier()

        def u_body(u, carry):
            for k in range(IB):
                c = u * IB + k
                b = k % NBUF
                s = k % SBUF
                _gather_wait(k, b)
                if k < SBUF:
                    @pl.when(u > 0)
                    def _():
                        _scatter_wait((k - SBUF) % IB, s)
                else:
                    _scatter_wait(k - SBUF, s)

                @plsc.parallel_loop(0, CHUNK, unroll=2)
                def _scale(e):
                    cvec = plsc.load_gather(
                        cbuf[k], [jnp.full((LANES,), e, jnp.int32)])
                    for m in range(d_feat // (2 * LANES)):
                        v32 = rows[b][e, pl.ds(m * LANES, LANES)]
                        v = plsc.bitcast(v32, jnp.bfloat16)
                        lo, hi = plsc.unpack(
                            v, format=plsc.PackFormat.INTERLEAVED,
                            preferred_element_type=jnp.float32)
                        srows[s][e, pl.ds(m * 2 * LANES, LANES)] = lo * cvec
                        srows[s][e, pl.ds(m * 2 * LANES + LANES, LANES)] = (
                            hi * cvec)

                _scatter_start(k, s)

                k3 = (k + 3) % IB
                b3 = (b + 3) % NBUF
                if k < IB - 3:
                    _idx_wait(c + 3, k3)
                    _gather_start(k3, b3)
                else:
                    @pl.when(u < nu - 1)
                    def _():
                        _idx_wait(c + 3, k3)
                        _gather_start(k3, b3)
                k6 = (k + 6) % IB
                if k < 2:
                    _idx_start(c + 6, k6)
                else:
                    @pl.when(u < nu - 1)
                    def _():
                        _idx_start(c + 6, k6)
            return carry

        lax.fori_loop(0, nu, u_body, 0)
        _scatter_wait((cpw - 2) % IB, (cpw - 2) % SBUF)
        _scatter_wait((cpw - 1) % IB, (cpw - 1) % SBUF)
        plsc.subcore_barrier()
        pltpu.sync_copy(acc.at[pl.ds(row0, rows_per_tile)],
                        out_hbm.at[cid, pl.ds(row0, rows_per_tile)])

    return sc_kernel


def _pack_body(x_ref, o_ref):
    x = x_ref[...]
    lo = jnp.concatenate([x[:, 32 * m:32 * m + 16] for m in range(4)], axis=1)
    hi = jnp.concatenate(
        [x[:, 32 * m + 16:32 * m + 32] for m in range(4)], axis=1)
    lob = jax.lax.bitcast_convert_type(
        lo.astype(jnp.bfloat16), jnp.uint16).astype(jnp.int32)
    hib = jax.lax.bitcast_convert_type(
        hi.astype(jnp.bfloat16), jnp.uint16).astype(jnp.int32)
    o_ref[...] = lob | (hib << 16)


def _pack_table(trace2d, n_nodes, d_feat):
    return pl.pallas_call(
        _pack_body,
        grid=(10,),
        out_shape=jax.ShapeDtypeStruct((n_nodes, d_feat // 2), jnp.int32),
        in_specs=[pl.BlockSpec((n_nodes // 10, d_feat), lambda i: (i, 0))],
        out_specs=pl.BlockSpec((n_nodes // 10, d_feat // 2), lambda i: (i, 0)),
    )(trace2d)


def _combine_body(g_ref, k_ref, p_ref, w_ref, o_ref):
    o_ref[...] = (g_ref[...] * w_ref[0] + k_ref[...] * w_ref[1]
                  + p_ref[0] + p_ref[1] + w_ref[2])


def _combine(gen2d, kill2d, partials, wvec, n_nodes, d_feat):
    blk = n_nodes // 10
    spec = pl.BlockSpec((blk, d_feat), lambda i: (i, 0))
    pspec = pl.BlockSpec((2, blk, d_feat), lambda i: (0, i, 0))
    return pl.pallas_call(
        _combine_body,
        grid=(10,),
        out_shape=jax.ShapeDtypeStruct((n_nodes, d_feat), jnp.float32),
        in_specs=[spec, spec, pspec,
                  pl.BlockSpec(memory_space=pltpu.SMEM)],
        out_specs=spec,
    )(gen2d, kill2d, partials, wvec)


def kernel(cfg_indices_padded, trace_h_i, gen_vectors, kill_vectors,
           direction, cfg_edges, W_edge, b_edge, W_upd, b_upd):
    b, n_nodes, d_feat = trace_h_i.shape
    n_edges = cfg_indices_padded.shape[1]
    assert b == 1

    quantum = NW * CHUNK * IB
    e_pad = -(-n_edges // quantum) * quantum
    n_pad = -(-n_nodes // (8 * NS)) * (8 * NS)

    w_coeff = jnp.stack(
        [W_edge[0, 0], W_edge[1, 0], b_edge[0], W_upd[2, 0]]).astype(jnp.float32)
    coeff = _edge_coeff(direction[0].reshape(1, n_edges).astype(jnp.float32),
                        cfg_edges[0].reshape(1, n_edges).astype(jnp.float32),
                        w_coeff, n_edges, e_pad)
    cfg2 = cfg_indices_padded.astype(jnp.int32).reshape(n_edges // 64, 128)
    src, tgt = _edge_idx(cfg2, n_edges, n_nodes, n_pad, e_pad)

    trace_i32 = _pack_table(trace_h_i[0], n_nodes, d_feat)
    zeros_hbm = jnp.zeros((n_pad, d_feat), jnp.float32)
    partials = _make_sc_kernel(n_pad, d_feat, e_pad)(
        trace_i32, src, tgt, coeff, zeros_hbm)

    out2d = _combine(
        gen_vectors[0], kill_vectors[0], partials,
        jnp.stack([W_upd[0, 0], W_upd[1, 0], b_upd[0]]).astype(jnp.float32),
        n_nodes, d_feat)
    return out2d.reshape(1, n_nodes, d_feat, 1)


# speedup vs baseline: 2.2923x; 2.2923x over previous
import functools

import jax
import jax.numpy as jnp
from jax import lax
from jax.experimental import pallas as pl
from jax.experimental.pallas import tpu as pltpu
from jax.experimental.pallas import tpu_sc as plsc

NC = 2
NS = 16
LANES = 16
NW = NC * NS


def _coeff_body(d_ref, c_ref, w_ref, o_ref):
    o_ref[...] = (d_ref[...] * w_ref[0] + c_ref[...] * w_ref[1] + w_ref[2]) * w_ref[3]


def _edge_coeff(direction2d, cfg2d, wvec):
    return pl.pallas_call(
        _coeff_body,
        out_shape=jax.ShapeDtypeStruct(direction2d.shape, jnp.float32),
        in_specs=[
            pl.BlockSpec(memory_space=pltpu.VMEM),
            pl.BlockSpec(memory_space=pltpu.VMEM),
            pl.BlockSpec(memory_space=pltpu.SMEM),
        ],
        out_specs=pl.BlockSpec(memory_space=pltpu.VMEM),
    )(direction2d, cfg2d, wvec)


CHUNK = 64
NBUF = 4
SBUF = 2
IB = 8


def _make_sc_kernel(n_pad, d_feat, e_pad):
    cpw = e_pad // (NW * CHUNK)
    nu = cpw // IB
    rows_per_tile = n_pad // NS
    mesh = plsc.VectorSubcoreMesh(
        core_axis_name="c", subcore_axis_name="s", num_cores=NC, num_subcores=NS
    )

    @functools.partial(
        pl.kernel,
        out_type=jax.ShapeDtypeStruct((NC, n_pad, d_feat), jnp.float32),
        mesh=mesh,
        compiler_params=pltpu.CompilerParams(needs_layout_passes=False, use_tc_tiling_on_sc=False),
        scratch_types=[
            pltpu.VMEM_SHARED((n_pad, d_feat), jnp.float32),
            [pltpu.VMEM((CHUNK,), jnp.int32) for _ in range(IB)],
            [pltpu.VMEM((CHUNK,), jnp.int32) for _ in range(IB)],
            [pltpu.VMEM((CHUNK,), jnp.float32) for _ in range(IB)],
            [pltpu.VMEM((CHUNK, d_feat // 2), jnp.int32) for _ in range(NBUF)],
            [pltpu.VMEM((CHUNK, d_feat), jnp.float32) for _ in range(SBUF)],
            [pltpu.SemaphoreType.DMA for _ in range(NBUF)],
            [pltpu.SemaphoreType.DMA for _ in range(SBUF)],
            [pltpu.SemaphoreType.DMA for _ in range(IB)],
            pltpu.SemaphoreType.DMA,
        ],
    )
    def sc_kernel(trace_hbm, src_hbm, tgt_hbm, coeff_hbm, zeros_hbm,
                  out_hbm, acc, sidx, tidx, cbuf, rows, srows,
                  gsem, ssem, isem, zsem):
        cid = lax.axis_index("c")
        sid = lax.axis_index("s")
        wid = cid * NS + sid
        c0 = wid * cpw

        def _idx_start(c, sl):
            pltpu.async_copy(src_hbm.at[c0 + c], sidx[sl], isem[sl])
            pltpu.async_copy(tgt_hbm.at[c0 + c], tidx[sl], isem[sl])
            pltpu.async_copy(coeff_hbm.at[c0 + c], cbuf[sl], isem[sl])

        def _idx_wait(c, sl):
            pltpu.make_async_copy(src_hbm.at[c0 + c], sidx[sl], isem[sl]).wait()
            pltpu.make_async_copy(tgt_hbm.at[c0 + c], tidx[sl], isem[sl]).wait()
            pltpu.make_async_copy(coeff_hbm.at[c0 + c], cbuf[sl], isem[sl]).wait()

        def _gather_start(sl, b):
            pltpu.async_copy(trace_hbm.at[sidx[sl]], rows[b], gsem[b])

        def _gather_wait(sl, b):
            pltpu.make_async_copy(
                trace_hbm.at[sidx[sl]], rows[b], gsem[b]).wait()

        def _scatter_start(sl, s):
            pltpu.async_copy(srows[s], acc.at[tidx[sl]], ssem[s], add=True)

        def _scatter_wait(sl, s):
            pltpu.make_async_copy(srows[s], acc.at[tidx[sl]], ssem[s]).wait()

        row0 = sid * rows_per_tile
        for c in range(6):
            _idx_start(c, c)
        for c in range(3):
            _idx_wait(c, c)
            _gather_start(c, c)

        pltpu.async_copy(zeros_hbm.at[pl.ds(row0, rows_per_tile)],
                         acc.at[pl.ds(row0, rows_per_tile)], zsem)
        pltpu.make_async_copy(zeros_hbm.at[pl.ds(row0, rows_per_tile)],
                              acc.at[pl.ds(row0, rows_per_tile)], zsem).wait()
        plsc.subcore_barrier()

        def u_body(u, carry):
            for k in range(IB):
                c = u * IB + k
                b = k % NBUF
                s = k % SBUF
                _gather_wait(k, b)
                if k < SBUF:
                    @pl.when(u > 0)
                    def _():
                        _scatter_wait((k - SBUF) % IB, s)
                else:
                    _scatter_wait(k - SBUF, s)

                @plsc.parallel_loop(0, CHUNK, unroll=2)
                def _scale(e):
                    cvec = plsc.load_gather(
                        cbuf[k], [jnp.full((LANES,), e, jnp.int32)])
                    for m in range(d_feat // (2 * LANES)):
                        v32 = rows[b][e, pl.ds(m * LANES, LANES)]
                        v = plsc.bitcast(v32, jnp.bfloat16)
                        lo, hi = plsc.unpack(
                            v, format=plsc.PackFormat.INTERLEAVED,
                            preferred_element_type=jnp.float32)
                        srows[s][e, pl.ds(m * 2 * LANES, LANES)] = lo * cvec
                        srows[s][e, pl.ds(m * 2 * LANES + LANES, LANES)] = (
                            hi * cvec)

                _scatter_start(k, s)

                k3 = (k + 3) % IB
                b3 = (b + 3) % NBUF
                if k < IB - 3:
                    _idx_wait(c + 3, k3)
                    _gather_start(k3, b3)
                else:
                    @pl.when(u < nu - 1)
                    def _():
                        _idx_wait(c + 3, k3)
                        _gather_start(k3, b3)
                k6 = (k + 6) % IB
                if k < 2:
                    _idx_start(c + 6, k6)
                else:
                    @pl.when(u < nu - 1)
                    def _():
                        _idx_start(c + 6, k6)
            return carry

        lax.fori_loop(0, nu, u_body, 0)
        _scatter_wait((cpw - 2) % IB, (cpw - 2) % SBUF)
        _scatter_wait((cpw - 1) % IB, (cpw - 1) % SBUF)
        plsc.subcore_barrier()
        pltpu.sync_copy(acc.at[pl.ds(row0, rows_per_tile)],
                        out_hbm.at[cid, pl.ds(row0, rows_per_tile)])

    return sc_kernel


def _pack_body(x_ref, o_ref):
    x = x_ref[...]
    lo = jnp.concatenate([x[:, 32 * m:32 * m + 16] for m in range(4)], axis=1)
    hi = jnp.concatenate(
        [x[:, 32 * m + 16:32 * m + 32] for m in range(4)], axis=1)
    lob = jax.lax.bitcast_convert_type(
        lo.astype(jnp.bfloat16), jnp.uint16).astype(jnp.int32)
    hib = jax.lax.bitcast_convert_type(
        hi.astype(jnp.bfloat16), jnp.uint16).astype(jnp.int32)
    o_ref[...] = lob | (hib << 16)


def _pack_table(trace2d, n_nodes, d_feat):
    return pl.pallas_call(
        _pack_body,
        grid=(10,),
        out_shape=jax.ShapeDtypeStruct((n_nodes, d_feat // 2), jnp.int32),
        in_specs=[pl.BlockSpec((n_nodes // 10, d_feat), lambda i: (i, 0))],
        out_specs=pl.BlockSpec((n_nodes // 10, d_feat // 2), lambda i: (i, 0)),
    )(trace2d)


def _combine_body(g_ref, k_ref, p_ref, w_ref, o_ref):
    o_ref[...] = (g_ref[...] * w_ref[0] + k_ref[...] * w_ref[1]
                  + p_ref[0] + p_ref[1] + w_ref[2])


def _combine(gen2d, kill2d, partials, wvec, n_nodes, d_feat):
    blk = n_nodes // 10
    spec = pl.BlockSpec((blk, d_feat), lambda i: (i, 0))
    pspec = pl.BlockSpec((2, blk, d_feat), lambda i: (0, i, 0))
    return pl.pallas_call(
        _combine_body,
        grid=(10,),
        out_shape=jax.ShapeDtypeStruct((n_nodes, d_feat), jnp.float32),
        in_specs=[spec, spec, pspec,
                  pl.BlockSpec(memory_space=pltpu.SMEM)],
        out_specs=spec,
    )(gen2d, kill2d, partials, wvec)


def kernel(cfg_indices_padded, trace_h_i, gen_vectors, kill_vectors,
           direction, cfg_edges, W_edge, b_edge, W_upd, b_upd):
    b, n_nodes, d_feat = trace_h_i.shape
    n_edges = cfg_indices_padded.shape[1]
    assert b == 1

    idx = cfg_indices_padded[0].astype(jnp.int32)
    src = idx[:, 0]
    tgt = idx[:, 1]

    w_coeff = jnp.stack(
        [W_edge[0, 0], W_edge[1, 0], b_edge[0], W_upd[2, 0]]).astype(jnp.float32)
    rows = n_edges // CHUNK
    coeff = _edge_coeff(
        direction[0].reshape(rows, CHUNK).astype(jnp.float32),
        cfg_edges[0].reshape(rows, CHUNK).astype(jnp.float32),
        w_coeff,
    ).reshape(n_edges)

    quantum = NW * CHUNK * IB
    e_pad = -(-n_edges // quantum) * quantum
    pad = e_pad - n_edges
    n_pad = -(-n_nodes // (8 * NS)) * (8 * NS)
    if pad:
        r = jnp.arange(pad, dtype=jnp.int32)
        coeff = jnp.concatenate([coeff, jnp.zeros((pad,), jnp.float32)])
        src = jnp.concatenate([src, r % n_nodes])
        tgt = jnp.concatenate([tgt, n_nodes + r % (n_pad - n_nodes)])
    src = src.reshape(e_pad // CHUNK, CHUNK)
    tgt = tgt.reshape(e_pad // CHUNK, CHUNK)
    coeff = coeff.reshape(e_pad // CHUNK, CHUNK)

    trace_i32 = _pack_table(trace_h_i[0], n_nodes, d_feat)
    zeros_hbm = jnp.zeros((n_pad, d_feat), jnp.float32)
    partials = _make_sc_kernel(n_pad, d_feat, e_pad)(
        trace_i32, src, tgt, coeff, zeros_hbm)

    out2d = _combine(
        gen_vectors[0], kill_vectors[0], partials,
        jnp.stack([W_upd[0, 0], W_upd[1, 0], b_upd[0]]).astype(jnp.float32),
        n_nodes, d_feat)
    return out2d.reshape(1, n_nodes, d_feat, 1)
